# Initial kernel scaffold; baseline (speedup 1.0000x reference)
import jax
import jax.numpy as jnp
from jax.experimental import pallas as pl


def kernel(x, edge_index, batch, W1a, b1a, W1b, b1b, g1, be1, W2a, b2a, W2b, b2b, g2, be2, W3a, b3a, W3b, b3b, g3, be3, Wf1, bf1, Wf2, bf2):
    raise NotImplementedError("write your pallas kernel here")


# speedup vs baseline: 4.3351x; 4.3351x over previous
import functools

import jax
import jax.numpy as jnp
from jax import lax
from jax.experimental import pallas as pl
from jax.experimental.pallas import tpu as pltpu
from jax.experimental.pallas import tpu_sc as plsc

_N = 10000
_E = 160000
_G = 128
_BN = 1000
_NB = _N // _BN

_NS = 16
_K = 125
_CHA = _E // _NS // _K
_CHB = _E // (2 * _NS) // _K
_STR = 640
_NPAD = _NS * _STR


def _mesh():
    return plsc.VectorSubcoreMesh(core_axis_name="c", subcore_axis_name="s")

def _stripe_zero(acc, zrows, s):
    pltpu.sync_copy(zrows, acc.at[pl.ds(s * _STR, _STR)])


def _stripe_out(acc, o_hbm, s):
    tail = _N - (_NS - 1) * _STR

    @pl.when(s < _NS - 1)
    def _():
        pltpu.sync_copy(acc.at[pl.ds(s * _STR, _STR)],
                        o_hbm.at[pl.ds(s * _STR, _STR)])

    @pl.when(s == _NS - 1)
    def _():
        pltpu.sync_copy(acc.at[pl.ds((_NS - 1) * _STR, tail)],
                        o_hbm.at[pl.ds((_NS - 1) * _STR, tail)])


def _agg_loop(x_hbm, col0, src_v, dst_v, gbuf, acc, nchunk):
    def body(j, carry):
        pltpu.sync_copy(x_hbm.at[src_v.at[j], pl.ds(col0, 128)], gbuf)
        pltpu.sync_copy(gbuf, acc.at[dst_v.at[j]], add=True)
        return carry
    lax.fori_loop(0, nchunk, body, 0)


def _sc_agg384_builder():
    @functools.partial(
        pl.kernel,
        mesh=_mesh(),
        out_type=tuple(
            jax.ShapeDtypeStruct((_N, 128), jnp.float32) for _ in range(4)),
        scratch_types=[
            pltpu.VMEM((_CHA, _K), jnp.int32),
            pltpu.VMEM((_CHA, _K), jnp.int32),
            pltpu.VMEM((_CHB, _K), jnp.int32),
            pltpu.VMEM((_CHB, _K), jnp.int32),
            pltpu.VMEM((_K, 128), jnp.float32),
            pltpu.VMEM_SHARED((_NPAD, 128), jnp.float32),
        ],
    )
    def agg(xp, src16, dst16, src32, dst32, zrows, b0, b1, b2p0, b2p1,
            srcA, dstA, srcB, dstB, gbuf, acc):
        c = lax.axis_index("c")
        s = lax.axis_index("s")
        pltpu.sync_copy(src16.at[s], srcA)
        pltpu.sync_copy(dst16.at[s], dstA)
        pltpu.sync_copy(src32.at[c * _NS + s], srcB)
        pltpu.sync_copy(dst32.at[c * _NS + s], dstB)
        _stripe_zero(acc, zrows, s)
        plsc.subcore_barrier()

        @pl.when(c == 0)
        def _():
            _agg_loop(xp, 0, srcA, dstA, gbuf, acc, _CHA)

        @pl.when(c == 1)
        def _():
            _agg_loop(xp, 128, srcA, dstA, gbuf, acc, _CHA)

        plsc.subcore_barrier()

        @pl.when(c == 0)
        def _():
            _stripe_out(acc, b0, s)

        @pl.when(c == 1)
        def _():
            _stripe_out(acc, b1, s)

        plsc.subcore_barrier()
        _stripe_zero(acc, zrows, s)
        plsc.subcore_barrier()

        _agg_loop(xp, 256, srcB, dstB, gbuf, acc, _CHB)
        plsc.subcore_barrier()

        @pl.when(c == 0)
        def _():
            _stripe_out(acc, b2p0, s)

        @pl.when(c == 1)
        def _():
            _stripe_out(acc, b2p1, s)

    return agg


def _sc_agg256_builder():
    @functools.partial(
        pl.kernel,
        mesh=_mesh(),
        out_type=tuple(
            jax.ShapeDtypeStruct((_N, 128), jnp.float32) for _ in range(2)),
        scratch_types=[
            pltpu.VMEM((_CHA, _K), jnp.int32),
            pltpu.VMEM((_CHA, _K), jnp.int32),
            pltpu.VMEM((_K, 128), jnp.float32),
            pltpu.VMEM_SHARED((_NPAD, 128), jnp.float32),
        ],
    )
    def agg(h, src16, dst16, zrows, alo, ahi, srcA, dstA, gbuf, acc):
        c = lax.axis_index("c")
        s = lax.axis_index("s")
        pltpu.sync_copy(src16.at[s], srcA)
        pltpu.sync_copy(dst16.at[s], dstA)
        _stripe_zero(acc, zrows, s)
        plsc.subcore_barrier()

        @pl.when(c == 0)
        def _():
            _agg_loop(h, 0, srcA, dstA, gbuf, acc, _CHA)

        @pl.when(c == 1)
        def _():
            _agg_loop(h, 128, srcA, dstA, gbuf, acc, _CHA)

        plsc.subcore_barrier()

        @pl.when(c == 0)
        def _():
            _stripe_out(acc, alo, s)

        @pl.when(c == 1)
        def _():
            _stripe_out(acc, ahi, s)

    return agg


def _sc_agg128_builder():
    @functools.partial(
        pl.kernel,
        mesh=_mesh(),
        out_type=tuple(
            jax.ShapeDtypeStruct((_N, 128), jnp.float32) for _ in range(2)),
        scratch_types=[
            pltpu.VMEM((_CHB, _K), jnp.int32),
            pltpu.VMEM((_CHB, _K), jnp.int32),
            pltpu.VMEM((_K, 128), jnp.float32),
            pltpu.VMEM_SHARED((_NPAD, 128), jnp.float32),
        ],
    )
    def agg(h, src32, dst32, zrows, a0, a1, srcB, dstB, gbuf, acc):
        c = lax.axis_index("c")
        s = lax.axis_index("s")
        pltpu.sync_copy(src32.at[c * _NS + s], srcB)
        pltpu.sync_copy(dst32.at[c * _NS + s], dstB)
        _stripe_zero(acc, zrows, s)
        plsc.subcore_barrier()
        _agg_loop(h, 0, srcB, dstB, gbuf, acc, _CHB)
        plsc.subcore_barrier()

        @pl.when(c == 0)
        def _():
            _stripe_out(acc, a0, s)

        @pl.when(c == 1)
        def _():
            _stripe_out(acc, a1, s)

    return agg


def _aggregate384(xp, src16, dst16, src32, dst32, zrows):
    return _sc_agg384_builder()(xp, src16, dst16, src32, dst32, zrows)


def _aggregate256(h, src16, dst16, zrows):
    return _sc_agg256_builder()(h, src16, dst16, zrows)


def _aggregate128(h, src32, dst32, zrows):
    return _sc_agg128_builder()(h, src32, dst32, zrows)


def _bn_affine(st_ref, g_ref, be_ref):
    mean = st_ref[0:1, :] * (1.0 / _N)
    var = st_ref[1:2, :] * (1.0 / _N) - mean * mean
    scale = g_ref[...] / jnp.sqrt(var + 1e-5)
    shift = be_ref[...] - mean * scale
    return scale, shift


def _stats_update(st_ref, t, i):
    blk = jnp.concatenate(
        [jnp.sum(t, axis=0, keepdims=True),
         jnp.sum(t * t, axis=0, keepdims=True)], axis=0)

    @pl.when(i == 0)
    def _():
        st_ref[...] = blk

    @pl.when(i > 0)
    def _():
        st_ref[...] += blk


def _gin_mlp(z, wa_ref, ba_ref, wb_ref, bb_ref, t_ref, st_ref, i):
    u = jnp.dot(z, wa_ref[...], preferred_element_type=jnp.float32)
    u = jnp.maximum(u + ba_ref[...], 0.0)
    t = jnp.dot(u, wb_ref[...], preferred_element_type=jnp.float32)
    t = jnp.maximum(t + bb_ref[...], 0.0)
    t_ref[...] = t
    _stats_update(st_ref, t, i)


def _post1_kernel(xp_ref, b0_ref, b1_ref, p0_ref, p1_ref, wa_ref, ba_ref,
                  wb_ref, bb_ref, t_ref, st_ref):
    x = xp_ref[...]
    z = jnp.concatenate(
        [x[:, :128] + b0_ref[...],
         x[:, 128:256] + b1_ref[...],
         x[:, 256:] + p0_ref[...] + p1_ref[...]], axis=1)
    _gin_mlp(z, wa_ref, ba_ref, wb_ref, bb_ref, t_ref, st_ref,
             pl.program_id(0))


def _post2_kernel(h_ref, alo_ref, ahi_ref, wa_ref, ba_ref, wb_ref, bb_ref,
                  t_ref, st_ref):
    z = h_ref[...] + jnp.concatenate([alo_ref[...], ahi_ref[...]], axis=1)
    _gin_mlp(z, wa_ref, ba_ref, wb_ref, bb_ref, t_ref, st_ref,
             pl.program_id(0))


def _post3_kernel(h_ref, a0_ref, a1_ref, wa_ref, ba_ref, wb_ref, bb_ref,
                  t_ref, st_ref):
    z = h_ref[...] + a0_ref[...] + a1_ref[...]
    _gin_mlp(z, wa_ref, ba_ref, wb_ref, bb_ref, t_ref, st_ref,
             pl.program_id(0))


def _gin_post(body, h_parts, wa, ba, wb, bb):
    d = wa.shape[0]
    m = wb.shape[1]
    dm = wa.shape[1]
    in_specs = [
        pl.BlockSpec((_BN, p.shape[1]), lambda i: (i, 0)) for p in h_parts
    ] + [
        pl.BlockSpec((d, dm), lambda i: (0, 0)),
        pl.BlockSpec((1, dm), lambda i: (0, 0)),
        pl.BlockSpec((dm, m), lambda i: (0, 0)),
        pl.BlockSpec((1, m), lambda i: (0, 0)),
    ]
    return pl.pallas_call(
        body,
        grid=(_NB,),
        in_specs=in_specs,
        out_specs=[
            pl.BlockSpec((_BN, m), lambda i: (i, 0)),
            pl.BlockSpec((2, m), lambda i: (0, 0)),
        ],
        out_shape=[
            jax.ShapeDtypeStruct((_N, m), jnp.float32),
            jax.ShapeDtypeStruct((2, m), jnp.float32),
        ],
    )(*h_parts, wa, ba, wb, bb)


def _bnonly_kernel(t_ref, st_ref, g_ref, be_ref, o_ref):
    scale, shift = _bn_affine(st_ref, g_ref, be_ref)
    o_ref[...] = t_ref[...] * scale + shift


def _bnonly(t, st, g, be):
    n, k = t.shape
    return pl.pallas_call(
        _bnonly_kernel,
        grid=(_NB,),
        in_specs=[
            pl.BlockSpec((_BN, k), lambda i: (i, 0)),
            pl.BlockSpec((2, k), lambda i: (0, 0)),
            pl.BlockSpec((1, k), lambda i: (0, 0)),
            pl.BlockSpec((1, k), lambda i: (0, 0)),
        ],
        out_specs=pl.BlockSpec((_BN, k), lambda i: (i, 0)),
        out_shape=jax.ShapeDtypeStruct((n, k), jnp.float32),
    )(t, st, g, be)


def _pool_kernel(t_ref, st_ref, g_ref, be_ref, b_ref, wf1_ref, bf1_ref,
                 wf2_ref, bf2_ref, o_ref, p_ref):
    i = pl.program_id(0)

    @pl.when(i == 0)
    def _():
        p_ref[...] = jnp.zeros_like(p_ref)

    scale, shift = _bn_affine(st_ref, g_ref, be_ref)
    h = t_ref[...] * scale + shift
    b = b_ref[0]
    onehot = (lax.broadcasted_iota(jnp.int32, (_G, _BN), 0) == b
              ).astype(jnp.float32)
    p_ref[...] += jnp.dot(onehot, h, preferred_element_type=jnp.float32,
                          precision=lax.Precision.HIGHEST)

    @pl.when(i == _NB - 1)
    def _():
        p = p_ref[...]
        q = jnp.dot(p, wf1_ref[...], preferred_element_type=jnp.float32)
        q = jnp.maximum(q + bf1_ref[...], 0.0)
        r = jnp.dot(q, wf2_ref[...], preferred_element_type=jnp.float32)
        o_ref[...] = jnp.maximum(r + bf2_ref[...], 0.0)


def _pool_head(t, st, g, be, batch3, wf1, bf1, wf2, bf2):
    k = t.shape[1]
    return pl.pallas_call(
        _pool_kernel,
        grid=(_NB,),
        in_specs=[
            pl.BlockSpec((_BN, k), lambda i: (i, 0)),
            pl.BlockSpec((2, k), lambda i: (0, 0)),
            pl.BlockSpec((1, k), lambda i: (0, 0)),
            pl.BlockSpec((1, k), lambda i: (0, 0)),
            pl.BlockSpec((1, 1, _BN), lambda i: (i, 0, 0)),
            pl.BlockSpec((k, 16), lambda i: (0, 0)),
            pl.BlockSpec((1, 16), lambda i: (0, 0)),
            pl.BlockSpec((16, 1), lambda i: (0, 0)),
            pl.BlockSpec((1, 1), lambda i: (0, 0)),
        ],
        out_specs=pl.BlockSpec((_G, 1), lambda i: (0, 0)),
        out_shape=jax.ShapeDtypeStruct((_G, 1), jnp.float32),
        scratch_shapes=[pltpu.VMEM((_G, k), jnp.float32)],
    )(t, st, g, be, batch3, wf1, bf1, wf2, bf2)


def kernel(x, edge_index, batch, W1a, b1a, W1b, b1b, g1, be1, W2a, b2a, W2b,
           b2b, g2, be2, W3a, b3a, W3b, b3b, g3, be3, Wf1, bf1, Wf2, bf2):
    row = lambda v: v.reshape(1, -1)
    src16 = edge_index[0].reshape(_NS, _CHA, _K)
    dst16 = edge_index[1].reshape(_NS, _CHA, _K)
    src32 = edge_index[0].reshape(2 * _NS, _CHB, _K)
    dst32 = edge_index[1].reshape(2 * _NS, _CHB, _K)
    batch3 = batch.reshape(_NB, 1, _BN)
    zrows = jnp.zeros((_STR, 128), jnp.float32)

    xp = jnp.concatenate([x, jnp.zeros((_N, 384 - x.shape[1]), x.dtype)], 1)
    w1ap = jnp.concatenate(
        [W1a, jnp.zeros((384 - W1a.shape[0], W1a.shape[1]), W1a.dtype)], 0)

    b0, b1_, p0, p1 = _aggregate384(xp, src16, dst16, src32, dst32, zrows)
    t1, st1 = _gin_post(_post1_kernel, [xp, b0, b1_, p0, p1],
                        w1ap, row(b1a), W1b, row(b1b))

    h1 = _bnonly(t1, st1, row(g1), row(be1))
    alo, ahi = _aggregate256(h1, src16, dst16, zrows)
    t2, st2 = _gin_post(_post2_kernel, [h1, alo, ahi],
                        W2a, row(b2a), W2b, row(b2b))

    h2 = _bnonly(t2, st2, row(g2), row(be2))
    a0, a1 = _aggregate128(h2, src32, dst32, zrows)
    t3, st3 = _gin_post(_post3_kernel, [h2, a0, a1],
                        W3a, row(b3a), W3b, row(b3b))

    p = _pool_head(t3, st3, row(g3), row(be3), batch3, Wf1, row(bf1),
                   Wf2, row(bf2))
    return p.reshape(-1)


# speedup vs baseline: 5.4237x; 1.2511x over previous
import functools

import jax
import jax.numpy as jnp
from jax import lax
from jax.experimental import pallas as pl
from jax.experimental.pallas import tpu as pltpu
from jax.experimental.pallas import tpu_sc as plsc

_N = 10000
_E = 160000
_G = 128
_BN = 1000
_NB = _N // _BN

_NS = 16
_K = 125
_CHA = _E // _NS // _K
_CHB = _E // (2 * _NS) // _K
_STR = 632
_NPAD = _NS * _STR
_SEG = 40


def _mesh():
    return plsc.VectorSubcoreMesh(core_axis_name="c", subcore_axis_name="s")

def _stripe_zero(acc, zrows, s):
    pltpu.sync_copy(zrows, acc.at[pl.ds(s * _STR, _STR)])


def _stripe_out(acc, o_hbm, s):
    tail = _N - (_NS - 1) * _STR

    @pl.when(s < _NS - 1)
    def _():
        pltpu.sync_copy(acc.at[pl.ds(s * _STR, _STR)],
                        o_hbm.at[pl.ds(s * _STR, _STR)])

    @pl.when(s == _NS - 1)
    def _():
        pltpu.sync_copy(acc.at[pl.ds((_NS - 1) * _STR, tail)],
                        o_hbm.at[pl.ds((_NS - 1) * _STR, tail)])


def _agg_loop(x_hbm, col0, idxs_hbm, idxd_hbm, tid, sbuf, dbuf, gbufs,
              gsems, ssems, acc, nchunk):
    def gstart(j, b):
        pltpu.async_copy(x_hbm.at[sbuf.at[j], pl.ds(col0, 128)],
                         gbufs[b], gsems[b])

    def gwait(b):
        pltpu.make_async_copy(x_hbm.at[sbuf.at[0], pl.ds(col0, 128)],
                              gbufs[b], gsems[b]).wait()

    def sstart(j, b):
        pltpu.async_copy(gbufs[b], acc.at[dbuf.at[j]], ssems[b], add=True)

    def swait(b):
        pltpu.make_async_copy(gbufs[b], acc.at[dbuf.at[0]],
                              ssems[b]).wait()

    nseg = nchunk // _SEG

    def seg_body(sg, carry):
        pltpu.sync_copy(idxs_hbm.at[tid, pl.ds(sg * _SEG, _SEG)], sbuf)
        pltpu.sync_copy(idxd_hbm.at[tid, pl.ds(sg * _SEG, _SEG)], dbuf)
        gstart(0, 0)
        gstart(1, 1)

        def pair_body(i, carry2):
            j = 2 * i
            for b in range(2):
                gwait(b)
                sstart(j + b, b)
            for b in range(2):
                @pl.when(j + b + 2 < _SEG)
                def _(b=b, j=j):
                    swait(b)
                    gstart(j + b + 2, b)
            return carry2
        lax.fori_loop(0, _SEG // 2, pair_body, 0)
        swait(0)
        swait(1)
        return carry
    lax.fori_loop(0, nseg, seg_body, 0)


def _sc_agg384_builder():
    @functools.partial(
        pl.kernel,
        mesh=_mesh(),
        out_type=tuple(
            jax.ShapeDtypeStruct((_N, 128), jnp.float32) for _ in range(4)),
        scratch_types=[
            pltpu.VMEM((_SEG, _K), jnp.int32),
            pltpu.VMEM((_SEG, _K), jnp.int32),
            pltpu.VMEM((_K, 128), jnp.float32),
            pltpu.VMEM((_K, 128), jnp.float32),
            pltpu.VMEM_SHARED((_NPAD, 128), jnp.float32),
            pltpu.SemaphoreType.DMA,
            pltpu.SemaphoreType.DMA,
            pltpu.SemaphoreType.DMA,
            pltpu.SemaphoreType.DMA,
        ],
    )
    def agg(xp, src16, dst16, src32, dst32, zrows, b0, b1, b2p0, b2p1,
            sbuf, dbuf, g0, g1, acc, gs0, gs1, ss0, ss1):
        gbufs = [g0, g1]
        gsems = [gs0, gs1]
        ssems = [ss0, ss1]
        c = lax.axis_index("c")
        s = lax.axis_index("s")
        _stripe_zero(acc, zrows, s)
        plsc.subcore_barrier()

        @pl.when(c == 0)
        def _():
            _agg_loop(xp, 0, src16, dst16, s, sbuf, dbuf, gbufs, gsems,
                      ssems, acc, _CHA)

        @pl.when(c == 1)
        def _():
            _agg_loop(xp, 128, src16, dst16, s, sbuf, dbuf, gbufs, gsems,
                      ssems, acc, _CHA)

        plsc.subcore_barrier()

        @pl.when(c == 0)
        def _():
            _stripe_out(acc, b0, s)

        @pl.when(c == 1)
        def _():
            _stripe_out(acc, b1, s)

        plsc.subcore_barrier()
        _stripe_zero(acc, zrows, s)
        plsc.subcore_barrier()

        _agg_loop(xp, 256, src32, dst32, c * _NS + s, sbuf, dbuf, gbufs,
                  gsems, ssems, acc, _CHB)
        plsc.subcore_barrier()

        @pl.when(c == 0)
        def _():
            _stripe_out(acc, b2p0, s)

        @pl.when(c == 1)
        def _():
            _stripe_out(acc, b2p1, s)

    return agg


def _sc_agg256_builder():
    @functools.partial(
        pl.kernel,
        mesh=_mesh(),
        out_type=tuple(
            jax.ShapeDtypeStruct((_N, 128), jnp.float32) for _ in range(2)),
        scratch_types=[
            pltpu.VMEM((_SEG, _K), jnp.int32),
            pltpu.VMEM((_SEG, _K), jnp.int32),
            pltpu.VMEM((_K, 128), jnp.float32),
            pltpu.VMEM((_K, 128), jnp.float32),
            pltpu.VMEM_SHARED((_NPAD, 128), jnp.float32),
            pltpu.SemaphoreType.DMA,
            pltpu.SemaphoreType.DMA,
            pltpu.SemaphoreType.DMA,
            pltpu.SemaphoreType.DMA,
        ],
    )
    def agg(h, src16, dst16, zrows, alo, ahi, sbuf, dbuf, g0, g1,
            acc, gs0, gs1, ss0, ss1):
        gbufs = [g0, g1]
        gsems = [gs0, gs1]
        ssems = [ss0, ss1]
        c = lax.axis_index("c")
        s = lax.axis_index("s")
        _stripe_zero(acc, zrows, s)
        plsc.subcore_barrier()

        @pl.when(c == 0)
        def _():
            _agg_loop(h, 0, src16, dst16, s, sbuf, dbuf, gbufs, gsems,
                      ssems, acc, _CHA)

        @pl.when(c == 1)
        def _():
            _agg_loop(h, 128, src16, dst16, s, sbuf, dbuf, gbufs, gsems,
                      ssems, acc, _CHA)

        plsc.subcore_barrier()

        @pl.when(c == 0)
        def _():
            _stripe_out(acc, alo, s)

        @pl.when(c == 1)
        def _():
            _stripe_out(acc, ahi, s)

    return agg


def _sc_agg128_builder():
    @functools.partial(
        pl.kernel,
        mesh=_mesh(),
        out_type=tuple(
            jax.ShapeDtypeStruct((_N, 128), jnp.float32) for _ in range(2)),
        scratch_types=[
            pltpu.VMEM((_SEG, _K), jnp.int32),
            pltpu.VMEM((_SEG, _K), jnp.int32),
            pltpu.VMEM((_K, 128), jnp.float32),
            pltpu.VMEM((_K, 128), jnp.float32),
            pltpu.VMEM_SHARED((_NPAD, 128), jnp.float32),
            pltpu.SemaphoreType.DMA,
            pltpu.SemaphoreType.DMA,
            pltpu.SemaphoreType.DMA,
            pltpu.SemaphoreType.DMA,
        ],
    )
    def agg(h, src32, dst32, zrows, a0, a1, sbuf, dbuf, g0, g1,
            acc, gs0, gs1, ss0, ss1):
        gbufs = [g0, g1]
        gsems = [gs0, gs1]
        ssems = [ss0, ss1]
        c = lax.axis_index("c")
        s = lax.axis_index("s")
        _stripe_zero(acc, zrows, s)
        plsc.subcore_barrier()
        _agg_loop(h, 0, src32, dst32, c * _NS + s, sbuf, dbuf, gbufs,
                  gsems, ssems, acc, _CHB)
        plsc.subcore_barrier()

        @pl.when(c == 0)
        def _():
            _stripe_out(acc, a0, s)

        @pl.when(c == 1)
        def _():
            _stripe_out(acc, a1, s)

    return agg


def _aggregate384(xp, src16, dst16, src32, dst32, zrows):
    return _sc_agg384_builder()(xp, src16, dst16, src32, dst32, zrows)


def _aggregate256(h, src16, dst16, zrows):
    return _sc_agg256_builder()(h, src16, dst16, zrows)


def _aggregate128(h, src32, dst32, zrows):
    return _sc_agg128_builder()(h, src32, dst32, zrows)


def _bn_var_pass(t_ref, st_ref, vs_ref, p, i):
    mean = st_ref[0:1, :] * (1.0 / _N)
    dev = t_ref[...] - mean
    blk = jnp.sum(dev * dev, axis=0, keepdims=True)

    @pl.when(jnp.logical_and(p == 0, i == 0))
    def _():
        vs_ref[...] = blk

    @pl.when(jnp.logical_and(p == 0, i > 0))
    def _():
        vs_ref[...] += blk
    return mean


def _bn_apply(t_ref, mean, vs_ref, g_ref, be_ref):
    var = vs_ref[...] * (1.0 / _N)
    return (g_ref[...] * (t_ref[...] - mean) / jnp.sqrt(var + 1e-5)
            + be_ref[...])


def _stats_update(st_ref, t, i):
    blk = jnp.concatenate(
        [jnp.sum(t, axis=0, keepdims=True),
         jnp.sum(t * t, axis=0, keepdims=True)], axis=0)

    @pl.when(i == 0)
    def _():
        st_ref[...] = blk

    @pl.when(i > 0)
    def _():
        st_ref[...] += blk


def _gin_mlp(z, wa_ref, ba_ref, wb_ref, bb_ref, t_ref, st_ref, i):
    u = jnp.dot(z, wa_ref[...], preferred_element_type=jnp.float32)
    u = jnp.maximum(u + ba_ref[...], 0.0)
    t = jnp.dot(u, wb_ref[...], preferred_element_type=jnp.float32)
    t = jnp.maximum(t + bb_ref[...], 0.0)
    t_ref[...] = t
    _stats_update(st_ref, t, i)


def _post1_kernel(xp_ref, b0_ref, b1_ref, p0_ref, p1_ref, wa_ref, ba_ref,
                  wb_ref, bb_ref, t_ref, st_ref):
    x = xp_ref[...]
    z = jnp.concatenate(
        [x[:, :128] + b0_ref[...],
         x[:, 128:256] + b1_ref[...],
         x[:, 256:] + p0_ref[...] + p1_ref[...]], axis=1)
    _gin_mlp(z, wa_ref, ba_ref, wb_ref, bb_ref, t_ref, st_ref,
             pl.program_id(0))


def _post2_kernel(h_ref, alo_ref, ahi_ref, wa_ref, ba_ref, wb_ref, bb_ref,
                  t_ref, st_ref):
    z = h_ref[...] + jnp.concatenate([alo_ref[...], ahi_ref[...]], axis=1)
    _gin_mlp(z, wa_ref, ba_ref, wb_ref, bb_ref, t_ref, st_ref,
             pl.program_id(0))


def _post3_kernel(h_ref, a0_ref, a1_ref, wa_ref, ba_ref, wb_ref, bb_ref,
                  t_ref, st_ref):
    z = h_ref[...] + a0_ref[...] + a1_ref[...]
    _gin_mlp(z, wa_ref, ba_ref, wb_ref, bb_ref, t_ref, st_ref,
             pl.program_id(0))


def _gin_post(body, h_parts, wa, ba, wb, bb):
    d = wa.shape[0]
    m = wb.shape[1]
    dm = wa.shape[1]
    in_specs = [
        pl.BlockSpec((_BN, p.shape[1]), lambda i: (i, 0)) for p in h_parts
    ] + [
        pl.BlockSpec((d, dm), lambda i: (0, 0)),
        pl.BlockSpec((1, dm), lambda i: (0, 0)),
        pl.BlockSpec((dm, m), lambda i: (0, 0)),
        pl.BlockSpec((1, m), lambda i: (0, 0)),
    ]
    return pl.pallas_call(
        body,
        grid=(_NB,),
        in_specs=in_specs,
        out_specs=[
            pl.BlockSpec((_BN, m), lambda i: (i, 0)),
            pl.BlockSpec((2, m), lambda i: (0, 0)),
        ],
        out_shape=[
            jax.ShapeDtypeStruct((_N, m), jnp.float32),
            jax.ShapeDtypeStruct((2, m), jnp.float32),
        ],
    )(*h_parts, wa, ba, wb, bb)


def _pad_kernel(x_ref, o_ref):
    o_ref[...] = jnp.concatenate(
        [x_ref[...],
         jnp.zeros((x_ref.shape[0], o_ref.shape[1] - x_ref.shape[1]),
                   jnp.float32)], axis=1)


def _pad_cols(x, m):
    n, k = x.shape
    return pl.pallas_call(
        _pad_kernel,
        grid=(_NB,),
        in_specs=[pl.BlockSpec((_BN, k), lambda i: (i, 0))],
        out_specs=pl.BlockSpec((_BN, m), lambda i: (i, 0)),
        out_shape=jax.ShapeDtypeStruct((n, m), jnp.float32),
    )(x)


def _bnonly_kernel(t_ref, st_ref, g_ref, be_ref, o_ref, vs_ref):
    p = pl.program_id(0)
    i = pl.program_id(1)
    mean = _bn_var_pass(t_ref, st_ref, vs_ref, p, i)

    @pl.when(p == 1)
    def _():
        o_ref[...] = _bn_apply(t_ref, mean, vs_ref, g_ref, be_ref)


def _bnonly(t, st, g, be):
    n, k = t.shape
    return pl.pallas_call(
        _bnonly_kernel,
        grid=(2, _NB),
        in_specs=[
            pl.BlockSpec((_BN, k), lambda p, i: (i, 0)),
            pl.BlockSpec((2, k), lambda p, i: (0, 0)),
            pl.BlockSpec((1, k), lambda p, i: (0, 0)),
            pl.BlockSpec((1, k), lambda p, i: (0, 0)),
        ],
        out_specs=pl.BlockSpec((_BN, k), lambda p, i: (i, 0)),
        out_shape=jax.ShapeDtypeStruct((n, k), jnp.float32),
        scratch_shapes=[pltpu.VMEM((1, k), jnp.float32)],
    )(t, st, g, be)


def _pool_kernel(t_ref, st_ref, g_ref, be_ref, b_ref, wf1_ref, bf1_ref,
                 wf2_ref, bf2_ref, o_ref, p_ref, vs_ref):
    p = pl.program_id(0)
    i = pl.program_id(1)
    mean = _bn_var_pass(t_ref, st_ref, vs_ref, p, i)

    @pl.when(jnp.logical_and(p == 1, i == 0))
    def _():
        p_ref[...] = jnp.zeros_like(p_ref)

    @pl.when(p == 1)
    def _():
        h = _bn_apply(t_ref, mean, vs_ref, g_ref, be_ref)
        b = b_ref[0]
        onehot = (lax.broadcasted_iota(jnp.int32, (_G, _BN), 0) == b
                  ).astype(jnp.float32)
        p_ref[...] += jnp.dot(onehot, h, preferred_element_type=jnp.float32,
                              precision=lax.Precision.HIGHEST)

    @pl.when(jnp.logical_and(p == 1, i == _NB - 1))
    def _():
        pp = p_ref[...]
        q = jnp.dot(pp, wf1_ref[...], preferred_element_type=jnp.float32)
        q = jnp.maximum(q + bf1_ref[...], 0.0)
        r = jnp.dot(q, wf2_ref[...], preferred_element_type=jnp.float32)
        o_ref[...] = jnp.maximum(r + bf2_ref[...], 0.0)


def _pool_head(t, st, g, be, batch3, wf1, bf1, wf2, bf2):
    k = t.shape[1]
    return pl.pallas_call(
        _pool_kernel,
        grid=(2, _NB),
        in_specs=[
            pl.BlockSpec((_BN, k), lambda p, i: (i, 0)),
            pl.BlockSpec((2, k), lambda p, i: (0, 0)),
            pl.BlockSpec((1, k), lambda p, i: (0, 0)),
            pl.BlockSpec((1, k), lambda p, i: (0, 0)),
            pl.BlockSpec((1, 1, _BN), lambda p, i: (i, 0, 0)),
            pl.BlockSpec((k, 16), lambda p, i: (0, 0)),
            pl.BlockSpec((1, 16), lambda p, i: (0, 0)),
            pl.BlockSpec((16, 1), lambda p, i: (0, 0)),
            pl.BlockSpec((1, 1), lambda p, i: (0, 0)),
        ],
        out_specs=pl.BlockSpec((_G, 1), lambda p, i: (0, 0)),
        out_shape=jax.ShapeDtypeStruct((_G, 1), jnp.float32),
        scratch_shapes=[pltpu.VMEM((_G, k), jnp.float32),
                        pltpu.VMEM((1, k), jnp.float32)],
    )(t, st, g, be, batch3, wf1, bf1, wf2, bf2)


def kernel(x, edge_index, batch, W1a, b1a, W1b, b1b, g1, be1, W2a, b2a, W2b,
           b2b, g2, be2, W3a, b3a, W3b, b3b, g3, be3, Wf1, bf1, Wf2, bf2):
    row = lambda v: v.reshape(1, -1)
    src16 = edge_index[0].reshape(_NS, _CHA, _K)
    dst16 = edge_index[1].reshape(_NS, _CHA, _K)
    src32 = edge_index[0].reshape(2 * _NS, _CHB, _K)
    dst32 = edge_index[1].reshape(2 * _NS, _CHB, _K)
    batch3 = batch.reshape(_NB, 1, _BN)
    zrows = jnp.zeros((_STR, 128), jnp.float32)

    xp = _pad_cols(x, 384)
    w1ap = jnp.concatenate(
        [W1a, jnp.zeros((384 - W1a.shape[0], W1a.shape[1]), W1a.dtype)], 0)

    b0, b1_, p0, p1 = _aggregate384(xp, src16, dst16, src32, dst32, zrows)
    t1, st1 = _gin_post(_post1_kernel, [xp, b0, b1_, p0, p1],
                        w1ap, row(b1a), W1b, row(b1b))

    h1 = _bnonly(t1, st1, row(g1), row(be1))
    alo, ahi = _aggregate256(h1, src16, dst16, zrows)
    t2, st2 = _gin_post(_post2_kernel, [h1, alo, ahi],
                        W2a, row(b2a), W2b, row(b2b))

    h2 = _bnonly(t2, st2, row(g2), row(be2))
    a0, a1 = _aggregate128(h2, src32, dst32, zrows)
    t3, st3 = _gin_post(_post3_kernel, [h2, a0, a1],
                        W3a, row(b3a), W3b, row(b3b))

    p = _pool_head(t3, st3, row(g3), row(be3), batch3, Wf1, row(bf1),
                   Wf2, row(bf2))
    return p.reshape(-1)
